# baseline (speedup 1.0000x reference)
import functools

import numpy as np
import jax
import jax.numpy as jnp
from jax import lax
from jax.experimental import pallas as pl
from jax.experimental.pallas import tpu as pltpu
from jax.experimental.pallas import tpu_sc as plsc

_D = 3
_V = 5
_K = 12
_NDV = _D * _V
_Q = 8
_J = _K * _Q
_NW = 32
_LN2 = 0.6931471805599453

_t64, _glw64 = np.polynomial.legendre.leggauss(_Q)
_TQ2 = np.repeat(_t64, _K).astype(np.float32).reshape(1, _J)
_SEL2 = np.tile(np.eye(_K, dtype=np.float32), _Q)
_LF = np.zeros(128, np.float64)
_LF[1:] = np.cumsum(np.log(np.arange(1, 128.0)))
_LFP = np.stack([_LF, np.log(np.arange(1, 129, dtype=np.float64))]
                ).astype(np.float32)


def _prep_kernel(minp_ref, lenp_ref, wpre_ref, tq_ref, sel_ref,
                 ab_ref, hc_ref):
    f32 = jnp.float32
    minp = minp_ref[...]
    lenp = lenp_ref[...]
    x1 = jax.nn.sigmoid(minp)
    x2 = jax.nn.sigmoid(minp + jnp.exp(lenp))
    mid = (x1 + x2) * 0.5
    half = (x2 - x1) * 0.5
    sel = sel_ref[...]
    midj = lax.dot(mid, sel, preferred_element_type=f32)
    halfj = lax.dot(half, sel, preferred_element_type=f32)
    f = midj + halfj * tq_ref[...]
    lg1mf = jnp.log1p(-f)
    ab_ref[...] = jnp.concatenate([jnp.log(f) - lg1mf, lg1mf], axis=0)
    c1 = jax.nn.softmax(wpre_ref[...], axis=1) / (x2 - x1)
    hc_ref[...] = jnp.concatenate([half, c1], axis=0)


def _log_f32(z):
    f32, i32 = jnp.float32, jnp.int32
    bits = lax.bitcast_convert_type(z, i32)
    ex = lax.shift_right_logical(bits, 23) - 127
    man = lax.bitcast_convert_type(
        jnp.bitwise_or(jnp.bitwise_and(bits, 0x007FFFFF), 0x3F800000), f32)
    big = man > np.float32(1.4142135)
    man = jnp.where(big, man * 0.5, man)
    exf = (ex + jnp.where(big, jnp.ones((16,), i32),
                          jnp.zeros((16,), i32))).astype(f32)
    t = (man - 1.0) / (man + 1.0)
    t2 = t * t
    inner = 1.0 + t2 * (np.float32(1 / 3) + t2 * (np.float32(1 / 5)
            + t2 * (np.float32(1 / 7) + t2 * np.float32(1 / 9))))
    return 2.0 * t * inner + exf * np.float32(_LN2)


def _sc_body(inp_hbm, ab_hbm, hc_hbm, lfp_hbm, out_hbm,
             inp_v, ab_v, hc_v, lfp_v, out_v):
    f32, i32 = jnp.float32, jnp.int32
    wid = lax.axis_index("s") * 2 + lax.axis_index("c")
    per_w = inp_hbm.shape[1] // _NW
    base = wid * per_w
    pltpu.sync_copy(inp_hbm.at[:, pl.ds(base, per_w)], inp_v)
    pltpu.sync_copy(ab_hbm, ab_v)
    pltpu.sync_copy(hc_hbm, hc_v)
    pltpu.sync_copy(lfp_hbm, lfp_v)

    ngroups = per_w // 16

    def gbody(g, carry):
        off = g * 16
        vt = inp_v[0, pl.ds(off, 16)]
        dep = inp_v[1, pl.ds(off, 16)]
        alt = inp_v[2, pl.ds(off, 16)]
        one = jnp.ones((16,), i32)
        zero = jnp.zeros((16,), i32)
        db = jnp.where(dep >= 10, one, zero) + jnp.where(dep >= 20, one, zero)
        dv = db * _V + vt
        dv16 = dv + 16
        nf = dep.astype(f32)
        kf = alt.astype(f32)
        logc = (plsc.load_gather(lfp_v, [zero, dep])
                - plsc.load_gather(lfp_v, [zero, alt])
                - plsc.load_gather(lfp_v, [zero, dep - alt]))
        lnp1 = plsc.load_gather(lfp_v, [one, dep])

        ts = [jnp.zeros((16,), f32) for _ in range(_K)]
        for q in range(_Q):
            glw_q = np.float32(_glw64[q])
            for kc in range(_K):
                col = jnp.full((16,), q * _K + kc, i32)
                ak = plsc.load_gather(ab_v, [dv, col])
                bk = plsc.load_gather(ab_v, [dv16, col])
                ts[kc] = ts[kc] + glw_q * jnp.exp(kf * ak + nf * bk + logc)
        np1 = nf + 1.0
        z = jnp.zeros((16,), f32)
        for kc in range(_K):
            kcv = jnp.full((16,), kc, i32)
            halfg = plsc.load_gather(hc_v, [dv, kcv])
            c1g = plsc.load_gather(hc_v, [dv16, kcv])
            z = z + c1g * jnp.maximum(np1 * halfg * ts[kc], 1e-30)
        out_v[pl.ds(off, 16)] = _log_f32(z) - lnp1
        return carry

    lax.fori_loop(0, ngroups, gbody, 0)
    pltpu.sync_copy(out_v, out_hbm.at[pl.ds(base, per_w)])


@jax.jit
def kernel(variant_types_b, depths_b, alt_counts_b, weights_pre_softmax_dvk,
           min_pre_sigmoid_dvk, lengths_in_logit_space_pre_exp_dvk):
    f32 = jnp.float32
    bsz = variant_types_b.shape[0]
    per_w = bsz // _NW
    inp = jnp.stack([variant_types_b.astype(jnp.int32),
                     depths_b.astype(jnp.int32),
                     alt_counts_b.astype(jnp.int32)])
    pad16 = lambda a, val: jnp.concatenate(
        [a.reshape(_NDV, _K).astype(f32), jnp.full((1, _K), val, f32)], axis=0)
    minp = pad16(min_pre_sigmoid_dvk, -5.0)
    lenp = pad16(lengths_in_logit_space_pre_exp_dvk, 0.0)
    wpre = pad16(weights_pre_softmax_dvk, 0.0)

    full = lambda shape: pl.BlockSpec(shape, lambda: tuple(0 for _ in shape))
    ab_t, hc_t = pl.pallas_call(
        _prep_kernel,
        in_specs=[full((16, _K)), full((16, _K)), full((16, _K)),
                  full((1, _J)), full((_K, _J))],
        out_specs=[full((32, _J)), full((32, _K))],
        out_shape=[jax.ShapeDtypeStruct((32, _J), f32),
                   jax.ShapeDtypeStruct((32, _K), f32)],
    )(minp, lenp, wpre, jnp.asarray(_TQ2), jnp.asarray(_SEL2))
    ab_t = jnp.pad(ab_t, ((0, 0), (0, 1)))
    hc_t = jnp.pad(hc_t, ((0, 0), (0, 1)))

    sc_call = functools.partial(
        pl.kernel,
        mesh=plsc.VectorSubcoreMesh(core_axis_name="c", subcore_axis_name="s"),
        compiler_params=pltpu.CompilerParams(use_tc_tiling_on_sc=False,
                                             needs_layout_passes=False),
        out_type=jax.ShapeDtypeStruct((bsz,), f32),
        scratch_types=[
            pltpu.VMEM((3, per_w), jnp.int32),
            pltpu.VMEM((32, _J + 1), f32),
            pltpu.VMEM((32, _K + 1), f32),
            pltpu.VMEM((2, 128), f32),
            pltpu.VMEM((per_w,), f32),
        ],
    )(_sc_body)
    return sc_call(inp, ab_t, hc_t, jnp.asarray(_LFP))
